# baseline (device time: 20283 ns/iter reference)
import jax
import jax.numpy as jnp
from jax import lax
from jax.experimental import pallas as pl
from jax.experimental.pallas import tpu as pltpu

N_DEV = 4
NB = 4


def kernel(x, W1, W2):
    m, k = x.shape
    _, h_per = W1.shape
    _, n = W2.shape
    mb = m // NB

    def body(x_hbm, w1_hbm, w2_hbm, out_hbm,
             xf, w1f, w2f, xb_ref, w1b_ref, w2b_ref,
             accA, sendA, sendB, recvA, recvB, outv,
             load_sems, out_sems,
             sendA_sems, recvA_sems, sendB_sems, recvB_sems):
        p = lax.axis_index("i")
        nbrA = p ^ 1
        nbrB = 3 - p

        cp_x = pltpu.make_async_copy(x_hbm, xf, load_sems.at[0])
        cp_w1 = pltpu.make_async_copy(w1_hbm, w1f, load_sems.at[1])
        cp_w2 = pltpu.make_async_copy(w2_hbm, w2f, load_sems.at[2])
        cp_x.start()
        cp_w1.start()
        cp_w2.start()

        barrier_sem = pltpu.get_barrier_semaphore()
        for nbr in [nbrA, nbrB]:
            pl.semaphore_signal(
                barrier_sem, inc=1,
                device_id=(nbr,), device_id_type=pl.DeviceIdType.MESH,
            )

        cp_x.wait()
        xb_ref[:, :] = xf[:, :].astype(jnp.bfloat16)
        cp_w1.wait()
        w1b_ref[:, :] = w1f[:, :].astype(jnp.bfloat16)

        rdmaA = [
            pltpu.make_async_remote_copy(
                src_ref=sendA.at[b], dst_ref=recvA.at[b],
                send_sem=sendA_sems.at[b], recv_sem=recvA_sems.at[b],
                device_id=(nbrA,), device_id_type=pl.DeviceIdType.MESH,
            )
            for b in range(NB)
        ]
        rdmaB = [
            pltpu.make_async_remote_copy(
                src_ref=sendB.at[b], dst_ref=recvB.at[b],
                send_sem=sendB_sems.at[b], recv_sem=recvB_sems.at[b],
                device_id=(nbrB,), device_id_type=pl.DeviceIdType.MESH,
            )
            for b in range(NB)
        ]
        out_cps = [
            pltpu.make_async_copy(
                outv.at[b], out_hbm.at[b * mb:(b + 1) * mb, :], out_sems.at[b]
            )
            for b in range(NB)
        ]

        for b in range(NB):
            hb = jnp.maximum(
                jnp.dot(
                    xb_ref[b * mb:(b + 1) * mb, :], w1b_ref[:, :],
                    preferred_element_type=jnp.float32,
                ),
                0.0,
            ).astype(jnp.bfloat16)
            if b == 0:
                cp_w2.wait()
                w2b_ref[:, :] = w2f[:, :].astype(jnp.bfloat16)
            pb = jnp.dot(hb, w2b_ref[:, :], preferred_element_type=jnp.float32)
            accA[b] = pb
            sendA[b] = pb.astype(jnp.bfloat16)
            if b == 0:
                pl.semaphore_wait(barrier_sem, 2)
            rdmaA[b].start()

        for b in range(NB):
            rdmaA[b].wait_recv()
            sendB[b] = (accA[b] + recvA[b].astype(jnp.float32)).astype(
                jnp.bfloat16
            )
            rdmaB[b].start()
            if b >= 1:
                rdmaB[b - 1].wait_recv()
                outv[b - 1] = (
                    accA[b - 1]
                    + recvA[b - 1].astype(jnp.float32)
                    + recvB[b - 1].astype(jnp.float32)
                )
                out_cps[b - 1].start()
        rdmaB[NB - 1].wait_recv()
        outv[NB - 1] = (
            accA[NB - 1]
            + recvA[NB - 1].astype(jnp.float32)
            + recvB[NB - 1].astype(jnp.float32)
        )
        out_cps[NB - 1].start()

        for b in range(NB):
            rdmaA[b].wait_send()
            rdmaB[b].wait_send()
            out_cps[b].wait()

    return pl.pallas_call(
        body,
        out_shape=jax.ShapeDtypeStruct((m, n), jnp.float32),
        in_specs=[
            pl.BlockSpec(memory_space=pltpu.MemorySpace.HBM),
            pl.BlockSpec(memory_space=pltpu.MemorySpace.HBM),
            pl.BlockSpec(memory_space=pltpu.MemorySpace.HBM),
        ],
        out_specs=pl.BlockSpec(memory_space=pltpu.MemorySpace.HBM),
        scratch_shapes=[
            pltpu.VMEM((m, k), jnp.float32),
            pltpu.VMEM((k, h_per), jnp.float32),
            pltpu.VMEM((h_per, n), jnp.float32),
            pltpu.VMEM((m, k), jnp.bfloat16),
            pltpu.VMEM((k, h_per), jnp.bfloat16),
            pltpu.VMEM((h_per, n), jnp.bfloat16),
            pltpu.VMEM((NB, mb, n), jnp.float32),
            pltpu.VMEM((NB, mb, n), jnp.bfloat16),
            pltpu.VMEM((NB, mb, n), jnp.bfloat16),
            pltpu.VMEM((NB, mb, n), jnp.bfloat16),
            pltpu.VMEM((NB, mb, n), jnp.bfloat16),
            pltpu.VMEM((NB, mb, n), jnp.float32),
            pltpu.SemaphoreType.DMA((3,)),
            pltpu.SemaphoreType.DMA((NB,)),
            pltpu.SemaphoreType.DMA((NB,)),
            pltpu.SemaphoreType.DMA((NB,)),
            pltpu.SemaphoreType.DMA((NB,)),
            pltpu.SemaphoreType.DMA((NB,)),
        ],
        compiler_params=pltpu.CompilerParams(collective_id=0),
    )(x, W1, W2)


# device time: 18392 ns/iter; 1.1028x vs baseline; 1.1028x over previous
import jax
import jax.numpy as jnp
from jax import lax
from jax.experimental import pallas as pl
from jax.experimental.pallas import tpu as pltpu

N_DEV = 4
NB = 4


def kernel(x, W1, W2):
    m, k = x.shape
    _, h_per = W1.shape
    _, n = W2.shape
    mb = m // NB

    def body(xb_ref, w1b_ref, w2b_ref, out_ref,
             accA, sendA, sendB, recvA, recvB,
             sendA_sems, recvA_sems, sendB_sems, recvB_sems):
        p = lax.axis_index("i")
        nbrA = p ^ 1
        nbrB = 3 - p

        barrier_sem = pltpu.get_barrier_semaphore()
        for nbr in [nbrA, nbrB]:
            pl.semaphore_signal(
                barrier_sem, inc=1,
                device_id=(nbr,), device_id_type=pl.DeviceIdType.MESH,
            )

        rdmaA = [
            pltpu.make_async_remote_copy(
                src_ref=sendA.at[b], dst_ref=recvA.at[b],
                send_sem=sendA_sems.at[b], recv_sem=recvA_sems.at[b],
                device_id=(nbrA,), device_id_type=pl.DeviceIdType.MESH,
            )
            for b in range(NB)
        ]
        rdmaB = [
            pltpu.make_async_remote_copy(
                src_ref=sendB.at[b], dst_ref=recvB.at[b],
                send_sem=sendB_sems.at[b], recv_sem=recvB_sems.at[b],
                device_id=(nbrB,), device_id_type=pl.DeviceIdType.MESH,
            )
            for b in range(NB)
        ]

        for b in range(NB):
            hb = jnp.maximum(
                jnp.dot(
                    xb_ref[b * mb:(b + 1) * mb, :], w1b_ref[:, :],
                    preferred_element_type=jnp.float32,
                ),
                0.0,
            ).astype(jnp.bfloat16)
            pb = jnp.dot(hb, w2b_ref[:, :], preferred_element_type=jnp.float32)
            accA[b] = pb
            sendA[b] = pb.astype(jnp.bfloat16)
            if b == 0:
                pl.semaphore_wait(barrier_sem, 2)
            rdmaA[b].start()

        for b in range(NB):
            rdmaA[b].wait_recv()
            sendB[b] = (accA[b] + recvA[b].astype(jnp.float32)).astype(
                jnp.bfloat16
            )
            rdmaB[b].start()
            if b >= 1:
                rdmaB[b - 1].wait_recv()
                out_ref[(b - 1) * mb:b * mb, :] = (
                    accA[b - 1]
                    + recvA[b - 1].astype(jnp.float32)
                    + recvB[b - 1].astype(jnp.float32)
                )
        rdmaB[NB - 1].wait_recv()
        out_ref[(NB - 1) * mb:, :] = (
            accA[NB - 1]
            + recvA[NB - 1].astype(jnp.float32)
            + recvB[NB - 1].astype(jnp.float32)
        )

        for b in range(NB):
            rdmaA[b].wait_send()
            rdmaB[b].wait_send()

    call = pl.pallas_call(
        body,
        out_shape=jax.ShapeDtypeStruct((m, n), jnp.float32),
        in_specs=[
            pl.BlockSpec(memory_space=pltpu.VMEM),
            pl.BlockSpec(memory_space=pltpu.VMEM),
            pl.BlockSpec(memory_space=pltpu.VMEM),
        ],
        out_specs=pl.BlockSpec(memory_space=pltpu.VMEM),
        scratch_shapes=[
            pltpu.VMEM((NB, mb, n), jnp.float32),
            pltpu.VMEM((NB, mb, n), jnp.bfloat16),
            pltpu.VMEM((NB, mb, n), jnp.bfloat16),
            pltpu.VMEM((NB, mb, n), jnp.bfloat16),
            pltpu.VMEM((NB, mb, n), jnp.bfloat16),
            pltpu.SemaphoreType.DMA((NB,)),
            pltpu.SemaphoreType.DMA((NB,)),
            pltpu.SemaphoreType.DMA((NB,)),
            pltpu.SemaphoreType.DMA((NB,)),
        ],
        compiler_params=pltpu.CompilerParams(collective_id=0),
    )
    return call(
        x.astype(jnp.bfloat16),
        W1.astype(jnp.bfloat16),
        W2.astype(jnp.bfloat16),
    )
